# baseline (speedup 1.0000x reference)
import jax
import jax.numpy as jnp
from jax.experimental import pallas as pl
from jax.experimental.pallas import tpu as pltpu


def _body(a_ref, b_ref, x_ref, m_ref, o_ref):
    a = a_ref[0]
    b = b_ref[0]
    m = m_ref[...]
    scale = 1.0 + m * (a - 1.0)
    bias = m * b
    o_ref[...] = x_ref[...] * scale[None, :] + bias[None, :]


def kernel(inputs, mask, alpha, beta):
    B, C = inputs.shape
    maskf = mask.astype(jnp.float32)
    blk = 2048
    return pl.pallas_call(
        _body,
        grid=(B // blk,),
        in_specs=[
            pl.BlockSpec(memory_space=pltpu.SMEM),
            pl.BlockSpec(memory_space=pltpu.SMEM),
            pl.BlockSpec((blk, C), lambda i: (i, 0)),
            pl.BlockSpec((C,), lambda i: (0,)),
        ],
        out_specs=pl.BlockSpec((blk, C), lambda i: (i, 0)),
        out_shape=jax.ShapeDtypeStruct((B, C), jnp.float32),
    )(alpha, beta, inputs, maskf)
